# baseline (speedup 1.0000x reference)
import functools

import jax
import jax.numpy as jnp
from jax import lax
from jax.experimental import pallas as pl
from jax.experimental.pallas import tpu as pltpu
from jax.experimental.pallas import tpu_sc as plsc

E = 8
IN_DIM = 1024
DIM = 2048
B = 4096
OUT_DIM = 3072

BM = 256
G = B // BM + E
P = G * BM


def _routing(idx):
    idx = idx.astype(jnp.int32)
    order = jnp.argsort(idx, stable=True).astype(jnp.int32)
    counts = jnp.bincount(idx, length=E)
    blocks = (counts + BM - 1) // BM
    ends_blk = jnp.cumsum(blocks)
    start_al = ((ends_blk - blocks) * BM).astype(jnp.int32)
    cum = (jnp.cumsum(counts) - counts).astype(jnp.int32)
    es = idx[order]
    adj = start_al - cum
    dst = jnp.arange(B, dtype=jnp.int32) + adj[es]
    gidx = jnp.zeros((P,), jnp.int32).at[dst].set(order)
    pos = jnp.zeros((B,), jnp.int32).at[order].set(dst)
    nv = jnp.sum(blocks).astype(jnp.int32)
    be = jnp.searchsorted(ends_blk, jnp.minimum(jnp.arange(G), nv - 1),
                          side="right")
    be = jnp.minimum(be, E - 1).astype(jnp.int32)
    return gidx, pos, jnp.concatenate([be, nv[None]])


def _gather_rows(table, idx):
    return jnp.take(table, idx, axis=0, mode="clip")


_NW = 32


def _sc_gather(tables, idx, nsplit=1):
    d = sum(t.shape[1] for t in tables)
    r = idx.shape[0]
    rpw = r // _NW
    bytes_per_row = d * 4
    ch = max(c for c in range(8, rpw + 1, 8)
             if rpw % c == 0 and c * bytes_per_row <= 131072)
    nch = rpw // ch
    ds = d // nsplit
    dtype = tables[0].dtype
    out_t = jax.ShapeDtypeStruct((r, ds), dtype)
    mesh = plsc.VectorSubcoreMesh(core_axis_name="c", subcore_axis_name="s")
    nt = len(tables)

    @functools.partial(
        pl.kernel, mesh=mesh,
        out_type=tuple(out_t for _ in range(nsplit)),
        scratch_types=[
            pltpu.VMEM((rpw,), jnp.int32),
            pltpu.VMEM((2, ch, d), dtype),
            pltpu.SemaphoreType.DMA((2,)),
        ],
    )
    def k(*refs):
        tabs = refs[:nt]
        idx_hbm = refs[nt]
        outs = refs[nt + 1:nt + 1 + nsplit]
        idx_v, rows_v, sems = refs[nt + 1 + nsplit:]
        wid = lax.axis_index("s") * 2 + lax.axis_index("c")
        base = wid * rpw
        pltpu.sync_copy(idx_hbm.at[pl.ds(base, rpw)], idx_v)

        def fire(c):
            buf = c % 2
            col = 0
            cps = []
            for t in tabs:
                td = t.shape[1]
                cps.append(pltpu.make_async_copy(
                    t.at[idx_v.at[pl.ds(c * ch, ch)]],
                    rows_v.at[buf, :, pl.ds(col, td)], sems.at[buf]))
                cps[-1].start()
                col += td
            return cps

        def drain(c, cps):
            buf = c % 2
            for cp in cps:
                cp.wait()
            for s in range(nsplit):
                pltpu.sync_copy(rows_v.at[buf, :, pl.ds(s * ds, ds)],
                                outs[s].at[pl.ds(base + c * ch, ch)])

        prev = fire(0)
        for c in range(1, nch):
            cur = fire(c)
            drain(c - 1, prev)
            prev = cur
        drain(nch - 1, prev)

    res = k(*tables, idx)
    return res if nsplit > 1 else res[0]


def _grouped_kernel(act, out_dtype, be_ref, x_ref, w_ref, b_ref, o_ref, wb_ref):
    g = pl.program_id(1)
    prev = be_ref[jnp.maximum(g - 1, 0)]

    @pl.when(jnp.logical_and(g < be_ref[G],
                             jnp.logical_or(g == 0, be_ref[g] != prev)))
    def _cast():
        wb_ref[...] = w_ref[0].astype(jnp.bfloat16)

    @pl.when(g < be_ref[G])
    def _compute():
        y = lax.dot_general(
            x_ref[...].astype(jnp.bfloat16), wb_ref[...],
            (((1,), (1,)), ((), ())),
            preferred_element_type=jnp.float32,
        )
        y = y + b_ref[0]
        if act == "relu":
            y = jnp.maximum(y, 0.0)
        o_ref[...] = y.astype(out_dtype)


def _grouped_matmul(x, w, b, be, act, nsplit=1, out_dtype=jnp.bfloat16):
    rows = x.shape[0]
    _, n, k = w.shape
    bn = n // nsplit
    b2 = b.reshape(E, 1, n)
    grid_spec = pltpu.PrefetchScalarGridSpec(
        num_scalar_prefetch=1,
        grid=(nsplit, rows // BM),
        in_specs=[
            pl.BlockSpec((BM, k), lambda j, g, be: (g, 0)),
            pl.BlockSpec((1, bn, k), lambda j, g, be: (be[g], j, 0)),
            pl.BlockSpec((1, 1, bn), lambda j, g, be: (be[g], 0, j)),
        ],
        out_specs=pl.BlockSpec((BM, bn), lambda j, g, be: (g, j)),
        scratch_shapes=[pltpu.VMEM((bn, k), jnp.bfloat16)],
    )
    return pl.pallas_call(
        functools.partial(_grouped_kernel, act, out_dtype),
        grid_spec=grid_spec,
        out_shape=jax.ShapeDtypeStruct((rows, n), out_dtype),
    )(be, x, w, b2)


def _dense_kernel(act, nv_ref, x_ref, w_ref, b_ref, o_ref, wb_ref):
    g = pl.program_id(0)

    @pl.when(g == 0)
    def _cast():
        wb_ref[...] = w_ref[...].astype(jnp.bfloat16)

    @pl.when(g < nv_ref[0])
    def _compute():
        y = lax.dot_general(
            x_ref[...], wb_ref[...],
            (((1,), (1,)), ((), ())),
            preferred_element_type=jnp.float32,
        )
        y = y + b_ref[...]
        if act == "relu":
            y = jnp.maximum(y, 0.0)
        else:
            y = jnp.tanh(y)
        o_ref[...] = y.astype(jnp.bfloat16)


def _dense(x, w, b, nv, act):
    rows = x.shape[0]
    n, k = w.shape
    grid_spec = pltpu.PrefetchScalarGridSpec(
        num_scalar_prefetch=1,
        grid=(rows // BM,),
        in_specs=[
            pl.BlockSpec((BM, k), lambda g, nv: (g, 0)),
            pl.BlockSpec((n, k), lambda g, nv: (0, 0)),
            pl.BlockSpec((1, n), lambda g, nv: (0, 0)),
        ],
        out_specs=pl.BlockSpec((BM, n), lambda g, nv: (g, 0)),
        scratch_shapes=[pltpu.VMEM((n, k), jnp.bfloat16)],
    )
    return pl.pallas_call(
        functools.partial(_dense_kernel, act),
        grid_spec=grid_spec,
        out_shape=jax.ShapeDtypeStruct((rows, n), jnp.bfloat16),
    )(nv, x, w, b.reshape(1, n))


def kernel(in_list, o, i, W1, b1, W2, b2, W3, b3, W4, b4, W5, b5):
    gidx_i, pos_i, be_i = _routing(i)
    gidx_o, pos_o, be_o = _routing(o)

    nv_i = be_i[G:]
    x_s = _sc_gather([in_list[0], in_list[1]], gidx_i)
    h = _grouped_matmul(x_s, W1, b1, be_i, act="relu")
    h = _dense(h, W2, b2, nv_i, "relu")
    h = _dense(h, W3, b3, nv_i, "relu")
    h = _dense(h, W4, b4, nv_i, "tanh")
    h = _gather_rows(h, pos_i[gidx_o])
    y = _grouped_matmul(h, W5, b5, be_o, act=None, nsplit=2,
                        out_dtype=jnp.float32)
    return _sc_gather([y], pos_o, nsplit=3)


# speedup vs baseline: 1.0812x; 1.0812x over previous
import functools

import jax
import jax.numpy as jnp
from jax import lax
from jax.experimental import pallas as pl
from jax.experimental.pallas import tpu as pltpu
from jax.experimental.pallas import tpu_sc as plsc

E = 8
IN_DIM = 1024
DIM = 2048
B = 4096
OUT_DIM = 3072

BM = 256
G = B // BM + E
P = G * BM


def _routing(idx):
    idx = idx.astype(jnp.int32)
    order = jnp.argsort(idx, stable=True).astype(jnp.int32)
    counts = jnp.bincount(idx, length=E)
    blocks = (counts + BM - 1) // BM
    ends_blk = jnp.cumsum(blocks)
    start_al = ((ends_blk - blocks) * BM).astype(jnp.int32)
    cum = (jnp.cumsum(counts) - counts).astype(jnp.int32)
    es = idx[order]
    adj = start_al - cum
    dst = jnp.arange(B, dtype=jnp.int32) + adj[es]
    gidx = jnp.zeros((P,), jnp.int32).at[dst].add(order)
    pos = jnp.zeros((B,), jnp.int32).at[order].add(dst)
    nv = jnp.sum(blocks).astype(jnp.int32)
    be = jnp.searchsorted(ends_blk, jnp.minimum(jnp.arange(G), nv - 1),
                          side="right")
    be = jnp.minimum(be, E - 1).astype(jnp.int32)
    return gidx, pos, jnp.concatenate([be, nv[None]])


def _gather_rows(table, idx):
    return jnp.take(table, idx, axis=0, mode="clip")


_NW = 32


def _sc_gather(tables, idx, nsplit=1):
    d = sum(t.shape[1] for t in tables)
    r = idx.shape[0]
    rpw = r // _NW
    bytes_per_row = d * 4
    ch = max(c for c in range(8, rpw + 1, 8)
             if rpw % c == 0 and c * bytes_per_row <= 131072)
    nch = rpw // ch
    ds = d // nsplit
    dtype = tables[0].dtype
    out_t = jax.ShapeDtypeStruct((r, ds), dtype)
    mesh = plsc.VectorSubcoreMesh(core_axis_name="c", subcore_axis_name="s")
    nt = len(tables)

    @functools.partial(
        pl.kernel, mesh=mesh,
        out_type=tuple(out_t for _ in range(nsplit)),
        scratch_types=[
            pltpu.VMEM((rpw,), jnp.int32),
            pltpu.VMEM((2, ch, d), dtype),
            pltpu.SemaphoreType.DMA((2,)),
        ],
    )
    def k(*refs):
        tabs = refs[:nt]
        idx_hbm = refs[nt]
        outs = refs[nt + 1:nt + 1 + nsplit]
        idx_v, rows_v, sems = refs[nt + 1 + nsplit:]
        wid = lax.axis_index("s") * 2 + lax.axis_index("c")
        base = wid * rpw
        pltpu.sync_copy(idx_hbm.at[pl.ds(base, rpw)], idx_v)

        def fire(c):
            buf = c % 2
            col = 0
            cps = []
            for t in tabs:
                td = t.shape[1]
                cps.append(pltpu.make_async_copy(
                    t.at[idx_v.at[pl.ds(c * ch, ch)]],
                    rows_v.at[buf, :, pl.ds(col, td)], sems.at[buf]))
                cps[-1].start()
                col += td
            return cps

        def drain(c, cps):
            buf = c % 2
            for cp in cps:
                cp.wait()
            for s in range(nsplit):
                pltpu.sync_copy(rows_v.at[buf, :, pl.ds(s * ds, ds)],
                                outs[s].at[pl.ds(base + c * ch, ch)])

        prev = fire(0)
        for c in range(1, nch):
            cur = fire(c)
            drain(c - 1, prev)
            prev = cur
        drain(nch - 1, prev)

    res = k(*tables, idx)
    return res if nsplit > 1 else res[0]


def _grouped_kernel(act, out_dtype, be_ref, x_ref, w_ref, b_ref, o_ref, wb_ref):
    g = pl.program_id(1)
    prev = be_ref[jnp.maximum(g - 1, 0)]

    @pl.when(jnp.logical_and(g < be_ref[G],
                             jnp.logical_or(g == 0, be_ref[g] != prev)))
    def _cast():
        wb_ref[...] = w_ref[0].astype(jnp.bfloat16)

    @pl.when(g < be_ref[G])
    def _compute():
        y = lax.dot_general(
            x_ref[...].astype(jnp.bfloat16), wb_ref[...],
            (((1,), (1,)), ((), ())),
            preferred_element_type=jnp.float32,
        )
        y = y + b_ref[0]
        if act == "relu":
            y = jnp.maximum(y, 0.0)
        o_ref[...] = y.astype(out_dtype)


def _grouped_matmul(x, w, b, be, act, nsplit=1, out_dtype=jnp.bfloat16):
    rows = x.shape[0]
    _, n, k = w.shape
    bn = n // nsplit
    b2 = b.reshape(E, 1, n)
    grid_spec = pltpu.PrefetchScalarGridSpec(
        num_scalar_prefetch=1,
        grid=(nsplit, rows // BM),
        in_specs=[
            pl.BlockSpec((BM, k), lambda j, g, be: (g, 0)),
            pl.BlockSpec((1, bn, k), lambda j, g, be: (be[g], j, 0)),
            pl.BlockSpec((1, 1, bn), lambda j, g, be: (be[g], 0, j)),
        ],
        out_specs=pl.BlockSpec((BM, bn), lambda j, g, be: (g, j)),
        scratch_shapes=[pltpu.VMEM((bn, k), jnp.bfloat16)],
    )
    return pl.pallas_call(
        functools.partial(_grouped_kernel, act, out_dtype),
        grid_spec=grid_spec,
        out_shape=jax.ShapeDtypeStruct((rows, n), out_dtype),
    )(be, x, w, b2)


def _dense_kernel(act, nv_ref, x_ref, w_ref, b_ref, o_ref, wb_ref):
    g = pl.program_id(0)

    @pl.when(g == 0)
    def _cast():
        wb_ref[...] = w_ref[...].astype(jnp.bfloat16)

    @pl.when(g < nv_ref[0])
    def _compute():
        y = lax.dot_general(
            x_ref[...], wb_ref[...],
            (((1,), (1,)), ((), ())),
            preferred_element_type=jnp.float32,
        )
        y = y + b_ref[...]
        if act == "relu":
            y = jnp.maximum(y, 0.0)
        else:
            y = jnp.tanh(y)
        o_ref[...] = y.astype(jnp.bfloat16)


def _dense(x, w, b, nv, act):
    rows = x.shape[0]
    n, k = w.shape
    grid_spec = pltpu.PrefetchScalarGridSpec(
        num_scalar_prefetch=1,
        grid=(rows // BM,),
        in_specs=[
            pl.BlockSpec((BM, k), lambda g, nv: (g, 0)),
            pl.BlockSpec((n, k), lambda g, nv: (0, 0)),
            pl.BlockSpec((1, n), lambda g, nv: (0, 0)),
        ],
        out_specs=pl.BlockSpec((BM, n), lambda g, nv: (g, 0)),
        scratch_shapes=[pltpu.VMEM((n, k), jnp.bfloat16)],
    )
    return pl.pallas_call(
        functools.partial(_dense_kernel, act),
        grid_spec=grid_spec,
        out_shape=jax.ShapeDtypeStruct((rows, n), jnp.bfloat16),
    )(nv, x, w, b.reshape(1, n))


def kernel(in_list, o, i, W1, b1, W2, b2, W3, b3, W4, b4, W5, b5):
    x = jnp.concatenate([in_list[0], in_list[1]], axis=1)
    gidx_i, pos_i, be_i = _routing(i)
    gidx_o, pos_o, be_o = _routing(o)

    nv_i = be_i[G:]
    x_s = _gather_rows(x.astype(jnp.bfloat16), gidx_i)
    h = _grouped_matmul(x_s, W1, b1, be_i, act="relu")
    h = _dense(h, W2, b2, nv_i, "relu")
    h = _dense(h, W3, b3, nv_i, "relu")
    h = _dense(h, W4, b4, nv_i, "tanh")
    h = _gather_rows(h, pos_i[gidx_o])
    y = _grouped_matmul(h, W5, b5, be_o, act=None, nsplit=6,
                        out_dtype=jnp.float32)
    return _sc_gather([y], pos_o, nsplit=3)


# speedup vs baseline: 1.1562x; 1.0693x over previous
import functools

import jax
import jax.numpy as jnp
from jax import lax
from jax.experimental import pallas as pl
from jax.experimental.pallas import tpu as pltpu
from jax.experimental.pallas import tpu_sc as plsc

E = 8
IN_DIM = 1024
DIM = 2048
B = 4096
OUT_DIM = 3072

BM = 256
G = B // BM + E
P = G * BM


def _routing(idx):
    idx = idx.astype(jnp.int32)
    order = jnp.argsort(idx, stable=True).astype(jnp.int32)
    counts = jnp.bincount(idx, length=E)
    blocks = (counts + BM - 1) // BM
    ends_blk = jnp.cumsum(blocks)
    start_al = ((ends_blk - blocks) * BM).astype(jnp.int32)
    cum = (jnp.cumsum(counts) - counts).astype(jnp.int32)
    es = idx[order]
    adj = start_al - cum
    dst = jnp.arange(B, dtype=jnp.int32) + adj[es]
    gidx = jnp.zeros((P,), jnp.int32).at[dst].add(order)
    pos = jnp.zeros((B,), jnp.int32).at[order].add(dst)
    nv = jnp.sum(blocks).astype(jnp.int32)
    be = jnp.searchsorted(ends_blk, jnp.minimum(jnp.arange(G), nv - 1),
                          side="right")
    be = jnp.minimum(be, E - 1).astype(jnp.int32)
    return gidx, pos, jnp.concatenate([be, nv[None]])


def _gather_rows(table, idx):
    return jnp.take(table, idx, axis=0, mode="clip")


_NW = 32


def _sc_gather(tables, idx, nsplit=1):
    d = sum(t.shape[1] for t in tables)
    r = idx.shape[0]
    rpw = r // _NW
    bytes_per_row = d * 4
    ch = max(c for c in range(8, rpw + 1, 8)
             if rpw % c == 0 and c * bytes_per_row <= 131072)
    nch = rpw // ch
    ds = d // nsplit
    dtype = tables[0].dtype
    out_t = jax.ShapeDtypeStruct((r, ds), dtype)
    mesh = plsc.VectorSubcoreMesh(core_axis_name="c", subcore_axis_name="s")
    nt = len(tables)

    @functools.partial(
        pl.kernel, mesh=mesh,
        out_type=tuple(out_t for _ in range(nsplit)),
        scratch_types=[
            pltpu.VMEM((rpw,), jnp.int32),
            pltpu.VMEM((2, ch, d), dtype),
            pltpu.SemaphoreType.DMA((2,)),
        ],
    )
    def k(*refs):
        tabs = refs[:nt]
        idx_hbm = refs[nt]
        outs = refs[nt + 1:nt + 1 + nsplit]
        idx_v, rows_v, sems = refs[nt + 1 + nsplit:]
        wid = lax.axis_index("s") * 2 + lax.axis_index("c")
        base = wid * rpw
        pltpu.sync_copy(idx_hbm.at[pl.ds(base, rpw)], idx_v)

        def fire(c):
            buf = c % 2
            col = 0
            cps = []
            for t in tabs:
                td = t.shape[1]
                cps.append(pltpu.make_async_copy(
                    t.at[idx_v.at[pl.ds(c * ch, ch)]],
                    rows_v.at[buf, :, pl.ds(col, td)], sems.at[buf]))
                cps[-1].start()
                col += td
            return cps

        def drain(c, cps):
            buf = c % 2
            for cp in cps:
                cp.wait()
            for s in range(nsplit):
                pltpu.sync_copy(rows_v.at[buf, :, pl.ds(s * ds, ds)],
                                outs[s].at[pl.ds(base + c * ch, ch)])

        prev = fire(0)
        for c in range(1, nch):
            cur = fire(c)
            drain(c - 1, prev)
            prev = cur
        drain(nch - 1, prev)

    res = k(*tables, idx)
    return res if nsplit > 1 else res[0]


def _grouped_kernel(act, out_dtype, be_ref, x_ref, w_ref, b_ref, o_ref, wb_ref):
    g = pl.program_id(1)
    prev = be_ref[jnp.maximum(g - 1, 0)]

    @pl.when(jnp.logical_and(g < be_ref[G],
                             jnp.logical_or(g == 0, be_ref[g] != prev)))
    def _cast():
        wb_ref[...] = w_ref[0].astype(jnp.bfloat16)

    @pl.when(g < be_ref[G])
    def _compute():
        y = lax.dot_general(
            x_ref[...].astype(jnp.bfloat16), wb_ref[...],
            (((1,), (1,)), ((), ())),
            preferred_element_type=jnp.float32,
        )
        y = y + b_ref[0]
        if act == "relu":
            y = jnp.maximum(y, 0.0)
        o_ref[...] = y.astype(out_dtype)


def _grouped_matmul(x, w, b, be, act, nsplit=1, out_dtype=jnp.bfloat16):
    rows = x.shape[0]
    _, n, k = w.shape
    bn = n // nsplit
    b2 = b.reshape(E, 1, n)
    grid_spec = pltpu.PrefetchScalarGridSpec(
        num_scalar_prefetch=1,
        grid=(nsplit, rows // BM),
        in_specs=[
            pl.BlockSpec((BM, k), lambda j, g, be: (g, 0)),
            pl.BlockSpec((1, bn, k), lambda j, g, be: (be[g], j, 0)),
            pl.BlockSpec((1, 1, bn), lambda j, g, be: (be[g], 0, j)),
        ],
        out_specs=pl.BlockSpec((BM, bn), lambda j, g, be: (g, j)),
        scratch_shapes=[pltpu.VMEM((bn, k), jnp.bfloat16)],
    )
    return pl.pallas_call(
        functools.partial(_grouped_kernel, act, out_dtype),
        grid_spec=grid_spec,
        out_shape=jax.ShapeDtypeStruct((rows, n), out_dtype),
    )(be, x, w, b2)


def _dense_kernel(act, nv_ref, x_ref, w_ref, b_ref, o_ref, wb_ref):
    g = pl.program_id(0)

    @pl.when(g == 0)
    def _cast():
        wb_ref[...] = w_ref[...].astype(jnp.bfloat16)

    @pl.when(g * (x_ref.shape[0] // BM) < nv_ref[0])
    def _compute():
        y = lax.dot_general(
            x_ref[...], wb_ref[...],
            (((1,), (1,)), ((), ())),
            preferred_element_type=jnp.float32,
        )
        y = y + b_ref[...]
        if act == "relu":
            y = jnp.maximum(y, 0.0)
        else:
            y = jnp.tanh(y)
        o_ref[...] = y.astype(jnp.bfloat16)


def _dense(x, w, b, nv, act, bm=512):
    rows = x.shape[0]
    n, k = w.shape
    grid_spec = pltpu.PrefetchScalarGridSpec(
        num_scalar_prefetch=1,
        grid=(rows // bm,),
        in_specs=[
            pl.BlockSpec((bm, k), lambda g, nv: (g, 0)),
            pl.BlockSpec((n, k), lambda g, nv: (0, 0)),
            pl.BlockSpec((1, n), lambda g, nv: (0, 0)),
        ],
        out_specs=pl.BlockSpec((bm, n), lambda g, nv: (g, 0)),
        scratch_shapes=[pltpu.VMEM((n, k), jnp.bfloat16)],
    )
    return pl.pallas_call(
        functools.partial(_dense_kernel, act),
        grid_spec=grid_spec,
        out_shape=jax.ShapeDtypeStruct((rows, n), jnp.bfloat16),
    )(nv, x, w, b.reshape(1, n))


def kernel(in_list, o, i, W1, b1, W2, b2, W3, b3, W4, b4, W5, b5):
    x = jnp.concatenate([in_list[0], in_list[1]], axis=1)
    gidx_i, pos_i, be_i = _routing(i)
    gidx_o, pos_o, be_o = _routing(o)

    nv_i = be_i[G:]
    x_s = _gather_rows(x.astype(jnp.bfloat16), gidx_i)
    h = _grouped_matmul(x_s, W1, b1, be_i, act="relu")
    h = _dense(h, W2, b2, nv_i, "relu")
    h = _dense(h, W3, b3, nv_i, "relu")
    h = _dense(h, W4, b4, nv_i, "tanh")
    h = _gather_rows(h, pos_i[gidx_o])
    y = _grouped_matmul(h, W5, b5, be_o, act=None, nsplit=3,
                        out_dtype=jnp.float32)
    return _sc_gather([y], pos_o, nsplit=3)


# speedup vs baseline: 1.1852x; 1.0251x over previous
import functools

import jax
import jax.numpy as jnp
from jax import lax
from jax.experimental import pallas as pl
from jax.experimental.pallas import tpu as pltpu
from jax.experimental.pallas import tpu_sc as plsc

E = 8
IN_DIM = 1024
DIM = 2048
B = 4096
OUT_DIM = 3072

BM = 256
G = B // BM + E
P = G * BM


def _routing(idx):
    idx = idx.astype(jnp.int32)
    order = jnp.argsort(idx, stable=True).astype(jnp.int32)
    counts = jnp.bincount(idx, length=E)
    blocks = (counts + BM - 1) // BM
    ends_blk = jnp.cumsum(blocks)
    start_al = ((ends_blk - blocks) * BM).astype(jnp.int32)
    cum = (jnp.cumsum(counts) - counts).astype(jnp.int32)
    es = idx[order]
    adj = start_al - cum
    dst = jnp.arange(B, dtype=jnp.int32) + adj[es]
    gidx = jnp.zeros((P,), jnp.int32).at[dst].add(order)
    pos = jnp.zeros((B,), jnp.int32).at[order].add(dst)
    nv = jnp.sum(blocks).astype(jnp.int32)
    be = jnp.searchsorted(ends_blk, jnp.minimum(jnp.arange(G), nv - 1),
                          side="right")
    be = jnp.minimum(be, E - 1).astype(jnp.int32)
    return gidx, pos, jnp.concatenate([be, nv[None]])


def _gather_rows(table, idx):
    return jnp.take(table, idx, axis=0, mode="clip")


_NW = 32


def _sc_gather(tables, idx, nsplit=1):
    d = sum(t.shape[1] for t in tables)
    r = idx.shape[0]
    rpw = r // _NW
    bytes_per_row = d * 4
    ch = max(c for c in range(8, rpw + 1, 8)
             if rpw % c == 0 and c * bytes_per_row <= 131072)
    nch = rpw // ch
    ds = d // nsplit
    dtype = tables[0].dtype
    out_t = jax.ShapeDtypeStruct((r, ds), dtype)
    mesh = plsc.VectorSubcoreMesh(core_axis_name="c", subcore_axis_name="s")
    nt = len(tables)

    @functools.partial(
        pl.kernel, mesh=mesh,
        out_type=tuple(out_t for _ in range(nsplit)),
        scratch_types=[
            pltpu.VMEM((rpw,), jnp.int32),
            pltpu.VMEM((2, ch, d), dtype),
            pltpu.SemaphoreType.DMA((2,)),
        ],
    )
    def k(*refs):
        tabs = refs[:nt]
        idx_hbm = refs[nt]
        outs = refs[nt + 1:nt + 1 + nsplit]
        idx_v, rows_v, sems = refs[nt + 1 + nsplit:]
        wid = lax.axis_index("s") * 2 + lax.axis_index("c")
        base = wid * rpw
        pltpu.sync_copy(idx_hbm.at[pl.ds(base, rpw)], idx_v)

        def fire(c):
            buf = c % 2
            col = 0
            cps = []
            for t in tabs:
                td = t.shape[1]
                cps.append(pltpu.make_async_copy(
                    t.at[idx_v.at[pl.ds(c * ch, ch)]],
                    rows_v.at[buf, :, pl.ds(col, td)], sems.at[buf]))
                cps[-1].start()
                col += td
            return cps

        def drain(c, cps):
            buf = c % 2
            for cp in cps:
                cp.wait()
            for s in range(nsplit):
                pltpu.sync_copy(rows_v.at[buf, :, pl.ds(s * ds, ds)],
                                outs[s].at[pl.ds(base + c * ch, ch)])

        prev = fire(0)
        for c in range(1, nch):
            cur = fire(c)
            drain(c - 1, prev)
            prev = cur
        drain(nch - 1, prev)

    res = k(*tables, idx)
    return res if nsplit > 1 else res[0]


def _grouped_kernel(act, out_dtype, be_ref, x_ref, w_ref, b_ref, o_ref, wb_ref):
    g = pl.program_id(1)
    prev = be_ref[jnp.maximum(g - 1, 0)]

    @pl.when(jnp.logical_and(g < be_ref[G],
                             jnp.logical_or(g == 0, be_ref[g] != prev)))
    def _cast():
        wb_ref[...] = w_ref[0].astype(jnp.bfloat16)

    @pl.when(g < be_ref[G])
    def _compute():
        y = lax.dot_general(
            x_ref[...].astype(jnp.bfloat16), wb_ref[...],
            (((1,), (1,)), ((), ())),
            preferred_element_type=jnp.float32,
        )
        y = y + b_ref[0]
        if act == "relu":
            y = jnp.maximum(y, 0.0)
        o_ref[...] = y.astype(out_dtype)


def _grouped_matmul(x, w, b, be, act, nsplit=1, out_dtype=jnp.bfloat16):
    rows = x.shape[0]
    _, n, k = w.shape
    bn = n // nsplit
    b2 = b.reshape(E, 1, n)
    grid_spec = pltpu.PrefetchScalarGridSpec(
        num_scalar_prefetch=1,
        grid=(nsplit, rows // BM),
        in_specs=[
            pl.BlockSpec((BM, k), lambda j, g, be: (g, 0)),
            pl.BlockSpec((1, bn, k), lambda j, g, be: (be[g], j, 0)),
            pl.BlockSpec((1, 1, bn), lambda j, g, be: (be[g], 0, j)),
        ],
        out_specs=pl.BlockSpec((BM, bn), lambda j, g, be: (g, j)),
        scratch_shapes=[pltpu.VMEM((bn, k), jnp.bfloat16)],
    )
    return pl.pallas_call(
        functools.partial(_grouped_kernel, act, out_dtype),
        grid_spec=grid_spec,
        out_shape=jax.ShapeDtypeStruct((rows, n), out_dtype),
    )(be, x, w, b2)


def _dense_kernel(act, nv_ref, x_ref, w_ref, b_ref, o_ref, wb_ref):
    g = pl.program_id(0)

    @pl.when(g == 0)
    def _cast():
        wb_ref[...] = w_ref[...].astype(jnp.bfloat16)

    @pl.when(g * (x_ref.shape[0] // BM) < nv_ref[0])
    def _compute():
        y = lax.dot_general(
            x_ref[...], wb_ref[...],
            (((1,), (1,)), ((), ())),
            preferred_element_type=jnp.float32,
        )
        y = y + b_ref[...]
        if act == "relu":
            y = jnp.maximum(y, 0.0)
        else:
            y = jnp.tanh(y)
        o_ref[...] = y.astype(jnp.bfloat16)


def _dense(x, w, b, nv, act, bm=512):
    rows = x.shape[0]
    n, k = w.shape
    grid_spec = pltpu.PrefetchScalarGridSpec(
        num_scalar_prefetch=1,
        grid=(rows // bm,),
        in_specs=[
            pl.BlockSpec((bm, k), lambda g, nv: (g, 0)),
            pl.BlockSpec((n, k), lambda g, nv: (0, 0)),
            pl.BlockSpec((1, n), lambda g, nv: (0, 0)),
        ],
        out_specs=pl.BlockSpec((bm, n), lambda g, nv: (g, 0)),
        scratch_shapes=[pltpu.VMEM((n, k), jnp.bfloat16)],
    )
    return pl.pallas_call(
        functools.partial(_dense_kernel, act),
        grid_spec=grid_spec,
        out_shape=jax.ShapeDtypeStruct((rows, n), jnp.bfloat16),
    )(nv, x, w, b.reshape(1, n))


def kernel(in_list, o, i, W1, b1, W2, b2, W3, b3, W4, b4, W5, b5):
    x = jnp.concatenate([in_list[0], in_list[1]], axis=1)
    gidx_i, pos_i, be_i = _routing(i)
    gidx_o, pos_o, be_o = _routing(o)

    nv_i = be_i[G:]
    x_s = _gather_rows(x.astype(jnp.bfloat16), gidx_i)
    h = _grouped_matmul(x_s, W1, b1, be_i, act="relu")
    h = _dense(h, W2, b2, nv_i, "relu")
    h = _dense(h, W3, b3, nv_i, "relu")
    h = _dense(h, W4, b4, nv_i, "tanh")
    h = _gather_rows(h, pos_i[gidx_o])
    y = _grouped_matmul(h, W5, b5, be_o, act=None, nsplit=2,
                        out_dtype=jnp.float32)
    return _sc_gather([y], pos_o, nsplit=3)
